# baseline (speedup 1.0000x reference)
import jax
import jax.numpy as jnp
from jax.experimental import pallas as pl
from jax.experimental.pallas import tpu as pltpu

_E_ROWS = 1_000_000
_DIM = 32
_FLAT = _E_ROWS * _DIM
_BLOCK = 1_280_000
_GRID = _FLAT // _BLOCK


def _copy_body(in_ref, out_ref):
    out_ref[...] = in_ref[...]


def kernel(embed):
    flat = embed.reshape(_FLAT)
    out = pl.pallas_call(
        _copy_body,
        grid=(_GRID,),
        in_specs=[pl.BlockSpec((_BLOCK,), lambda i: (i,))],
        out_specs=pl.BlockSpec((_BLOCK,), lambda i: (i,)),
        out_shape=jax.ShapeDtypeStruct((_FLAT,), jnp.float32),
    )(flat)
    return out.reshape(_E_ROWS, _DIM)


# speedup vs baseline: 13.7718x; 13.7718x over previous
import jax
import jax.numpy as jnp
from jax.experimental import pallas as pl
from jax.experimental.pallas import tpu as pltpu

_E_ROWS = 1_000_000
_DIM = 32
_BLOCK_COLS = 65536
_GRID = (_E_ROWS + _BLOCK_COLS - 1) // _BLOCK_COLS


def _copy_body(in_ref, out_ref):
    out_ref[...] = in_ref[...]


def kernel(embed):
    t = embed.T
    out = pl.pallas_call(
        _copy_body,
        grid=(_GRID,),
        in_specs=[pl.BlockSpec((_DIM, _BLOCK_COLS), lambda i: (0, i))],
        out_specs=pl.BlockSpec((_DIM, _BLOCK_COLS), lambda i: (0, i)),
        out_shape=jax.ShapeDtypeStruct((_DIM, _E_ROWS), jnp.float32),
    )(t)
    return out.T
